# Initial kernel scaffold; baseline (speedup 1.0000x reference)
import functools

import jax
import jax.numpy as jnp
from jax import lax
from jax.experimental import pallas as pl
from jax.experimental.pallas import tpu as pltpu
from jax.experimental.pallas import tpu_sc as plsc

N = 10000
E = 320000
D = 128
G = 64

NC = 2
NS = 16
NW = NC * NS
ET = E // NW
K = 80
NB = ET // K
ROWS_PER_TILE = N // NS

_mesh = plsc.VectorSubcoreMesh(core_axis_name="c", subcore_axis_name="s")


@functools.partial(
    pl.kernel,
    mesh=_mesh,
    out_type=jax.ShapeDtypeStruct((NC, N, 16), jnp.float32),
    scratch_types=[
        pltpu.VMEM_SHARED((N, 16), jnp.float32),
        pltpu.VMEM((K,), jnp.int32),
        pltpu.VMEM((K, 16), jnp.float32),
    ],
)
def _sc_degree(dst_hbm, zeros_hbm, ones_hbm, out_hbm, hist_sh, dst_v, ones_v):
    cid = lax.axis_index("c")
    sid = lax.axis_index("s")
    wid = cid * NS + sid
    r0 = sid * ROWS_PER_TILE
    pltpu.sync_copy(zeros_hbm.at[pl.ds(r0, ROWS_PER_TILE)],
                    hist_sh.at[pl.ds(r0, ROWS_PER_TILE)])
    pltpu.sync_copy(ones_hbm, ones_v)
    plsc.subcore_barrier()

    def body(j, carry):
        base = wid * ET + j * K
        pltpu.sync_copy(dst_hbm.at[pl.ds(base, K)], dst_v)
        pltpu.sync_copy(ones_v, hist_sh.at[dst_v], add=True)
        return carry

    lax.fori_loop(0, NB, body, 0)
    plsc.subcore_barrier()
    pltpu.sync_copy(hist_sh.at[pl.ds(r0, ROWS_PER_TILE)],
                    out_hbm.at[cid].at[pl.ds(r0, ROWS_PER_TILE)])


@functools.partial(
    pl.kernel,
    mesh=_mesh,
    out_type=jax.ShapeDtypeStruct((NC, N, D), jnp.float32),
    scratch_types=[
        pltpu.VMEM_SHARED((N, D), jnp.float32),
        pltpu.VMEM((K,), jnp.int32),
        pltpu.VMEM((K,), jnp.int32),
        pltpu.VMEM((K, D), jnp.float32),
        pltpu.SemaphoreType.DMA,
    ],
)
def _sc_scatter(y_hbm, zeros_hbm, src_hbm, dst_hbm, out_hbm,
                acc_sh, src_v, dst_v, rows_v, sem):
    cid = lax.axis_index("c")
    sid = lax.axis_index("s")
    wid = cid * NS + sid
    r0 = sid * ROWS_PER_TILE
    @pl.when(cid == 0)
    def _():
        pltpu.sync_copy(y_hbm.at[pl.ds(r0, ROWS_PER_TILE)],
                        acc_sh.at[pl.ds(r0, ROWS_PER_TILE)])

    @pl.when(cid != 0)
    def _():
        pltpu.sync_copy(zeros_hbm.at[pl.ds(r0, ROWS_PER_TILE)],
                        acc_sh.at[pl.ds(r0, ROWS_PER_TILE)])

    plsc.subcore_barrier()

    def body(j, carry):
        base = wid * ET + j * K
        pltpu.sync_copy(src_hbm.at[pl.ds(base, K)], src_v)
        pltpu.sync_copy(dst_hbm.at[pl.ds(base, K)], dst_v)
        pltpu.async_copy(y_hbm.at[src_v], rows_v, sem).wait()
        pltpu.sync_copy(rows_v, acc_sh.at[dst_v], add=True)
        return carry

    lax.fori_loop(0, NB, body, 0)
    plsc.subcore_barrier()
    pltpu.sync_copy(acc_sh.at[pl.ds(r0, ROWS_PER_TILE)],
                    out_hbm.at[cid].at[pl.ds(r0, ROWS_PER_TILE)])


_RB = 2000
_NRB = N // _RB


def _tc_prep_body(hist_ref, x_ref, w_ref, y_ref, dinv_ref):
    deg = hist_ref[0, :, 0:1] + hist_ref[1, :, 0:1] + 1.0
    dinv = lax.rsqrt(deg)
    dinv_ref[...] = dinv
    y_ref[...] = dinv * jnp.dot(x_ref[...], w_ref[...],
                                preferred_element_type=jnp.float32)


def _tc_prep(hist2, x, w1):
    return pl.pallas_call(
        _tc_prep_body,
        grid=(_NRB,),
        in_specs=[
            pl.BlockSpec((NC, _RB, 16), lambda j: (0, j, 0)),
            pl.BlockSpec((_RB, D), lambda j: (j, 0)),
            pl.BlockSpec((D, D), lambda j: (0, 0)),
        ],
        out_specs=[
            pl.BlockSpec((_RB, D), lambda j: (j, 0)),
            pl.BlockSpec((_RB, 1), lambda j: (j, 0)),
        ],
        out_shape=[
            jax.ShapeDtypeStruct((N, D), jnp.float32),
            jax.ShapeDtypeStruct((N, 1), jnp.float32),
        ],
    )(hist2, x, w1)


def _tc_mid_body(acc_ref, dinv_ref, b_ref, w_ref, y_ref):
    dinv = dinv_ref[...]
    h = dinv * (acc_ref[0] + acc_ref[1]) + b_ref[...]
    h = jnp.maximum(h, 0.0)
    y_ref[...] = dinv * jnp.dot(h, w_ref[...],
                                preferred_element_type=jnp.float32)


def _tc_mid(acc2, dinv, b, w_next):
    return pl.pallas_call(
        _tc_mid_body,
        grid=(_NRB,),
        in_specs=[
            pl.BlockSpec((NC, _RB, D), lambda j: (0, j, 0)),
            pl.BlockSpec((_RB, 1), lambda j: (j, 0)),
            pl.BlockSpec((1, D), lambda j: (0, 0)),
            pl.BlockSpec((D, D), lambda j: (0, 0)),
        ],
        out_specs=pl.BlockSpec((_RB, D), lambda j: (j, 0)),
        out_shape=jax.ShapeDtypeStruct((N, D), jnp.float32),
    )(acc2, dinv, b.reshape(1, D), w_next)


def _tc_final_body(acc_ref, dinv_ref, b_ref, batch_ref, out_ref, sums, cnt):
    j = pl.program_id(0)
    h3 = dinv_ref[...] * (acc_ref[0] + acc_ref[1]) + b_ref[...]
    batch = batch_ref[0, 0, :].reshape(1, _RB)
    gid = lax.broadcasted_iota(jnp.int32, (G, _RB), 0)
    p = (gid == batch).astype(jnp.float32)

    @pl.when(j == 0)
    def _():
        sums[...] = jnp.zeros_like(sums)
        cnt[...] = jnp.zeros_like(cnt)

    sums[...] += jnp.dot(p, h3, preferred_element_type=jnp.float32)
    cnt[...] += jnp.sum(p, axis=1, keepdims=True)

    @pl.when(j == _NRB - 1)
    def _():
        out_ref[...] = sums[...] / jnp.maximum(cnt[...], 1.0)


def _tc_final(acc2, dinv, b3, batch):
    return pl.pallas_call(
        _tc_final_body,
        grid=(_NRB,),
        in_specs=[
            pl.BlockSpec((NC, _RB, D), lambda j: (0, j, 0)),
            pl.BlockSpec((_RB, 1), lambda j: (j, 0)),
            pl.BlockSpec((1, D), lambda j: (0, 0)),
            pl.BlockSpec((1, 1, _RB), lambda j: (j, 0, 0)),
        ],
        out_specs=pl.BlockSpec((G, D), lambda j: (0, 0)),
        out_shape=jax.ShapeDtypeStruct((G, D), jnp.float32),
        scratch_shapes=[
            pltpu.VMEM((G, D), jnp.float32),
            pltpu.VMEM((G, 1), jnp.float32),
        ],
    )(acc2, dinv, b3.reshape(1, D), batch.reshape(_NRB, 1, _RB))


def kernel(x, edge_index, batch, W1, b1, W2, b2, W3, b3):
    src = edge_index[0]
    dst = edge_index[1]
    zeros_nd = jnp.zeros((N, D), jnp.float32)
    zeros_n16 = jnp.zeros((N, 16), jnp.float32)
    ones_k16 = jnp.ones((K, 16), jnp.float32)

    hist2 = _sc_degree(dst, zeros_n16, ones_k16)
    y1, dinv = _tc_prep(hist2, x, W1)
    acc1 = _sc_scatter(y1, zeros_nd, src, dst)
    y2 = _tc_mid(acc1, dinv, b1, W2)
    acc2 = _sc_scatter(y2, zeros_nd, src, dst)
    y3 = _tc_mid(acc2, dinv, b2, W3)
    acc3 = _sc_scatter(y3, zeros_nd, src, dst)
    return _tc_final(acc3, dinv, b3, batch)


# speedup vs baseline: 11.4702x; 11.4702x over previous
import functools

import jax
import jax.numpy as jnp
from jax import lax
from jax.experimental import pallas as pl
from jax.experimental.pallas import tpu as pltpu
from jax.experimental.pallas import tpu_sc as plsc

N = 10000
E = 320000
D = 128
G = 64

NC = 2
NS = 16
NW = NC * NS
ET = E // NW
K = 80
NB = ET // K
RCHUNK = 624

_mesh = plsc.VectorSubcoreMesh(core_axis_name="c", subcore_axis_name="s")


NP = 10240


@functools.partial(
    pl.kernel,
    mesh=_mesh,
    out_type=jax.ShapeDtypeStruct((NC, NP), jnp.float32),
    scratch_types=[
        pltpu.VMEM_SHARED((NP,), jnp.float32),
        pltpu.VMEM((K,), jnp.int32),
        pltpu.VMEM((K,), jnp.float32),
    ],
)
def _sc_degree(dst_hbm, zeros_hbm, ones_hbm, out_hbm, hist_sh, dst_v, ones_v):
    cid = lax.axis_index("c")
    sid = lax.axis_index("s")
    wid = cid * NS + sid
    r0 = sid * 640
    pltpu.sync_copy(zeros_hbm.at[pl.ds(r0, 640)],
                    hist_sh.at[pl.ds(r0, 640)])
    pltpu.sync_copy(ones_hbm, ones_v)
    plsc.subcore_barrier()

    def body(j, carry):
        base = wid * ET + j * K
        pltpu.sync_copy(dst_hbm.at[pl.ds(base, K)], dst_v)
        pltpu.sync_copy(ones_v, hist_sh.at[dst_v], add=True)
        return carry

    lax.fori_loop(0, NB, body, 0)
    plsc.subcore_barrier()
    pltpu.sync_copy(hist_sh.at[pl.ds(r0, 640)],
                    out_hbm.at[cid].at[pl.ds(r0, 640)])


@functools.partial(
    pl.kernel,
    mesh=_mesh,
    out_type=jax.ShapeDtypeStruct((NC, N, D), jnp.float32),
    scratch_types=[
        pltpu.VMEM_SHARED((N, D), jnp.float32),
        pltpu.VMEM((K,), jnp.int32),
        pltpu.VMEM((K,), jnp.int32),
        pltpu.VMEM((K, D), jnp.float32),
        pltpu.SemaphoreType.DMA,
    ],
)
def _sc_scatter(y_hbm, zeros_hbm, src_hbm, dst_hbm, out_hbm,
                acc_sh, src_v, dst_v, rows_v, sem):
    cid = lax.axis_index("c")
    sid = lax.axis_index("s")
    wid = cid * NS + sid
    r0 = sid * RCHUNK
    rtail0 = NS * RCHUNK
    ntail = N - rtail0
    @pl.when(cid == 0)
    def _():
        pltpu.sync_copy(y_hbm.at[pl.ds(r0, RCHUNK)],
                        acc_sh.at[pl.ds(r0, RCHUNK)])

        @pl.when(sid == NS - 1)
        def _():
            pltpu.sync_copy(y_hbm.at[pl.ds(rtail0, ntail)],
                            acc_sh.at[pl.ds(rtail0, ntail)])

    @pl.when(cid != 0)
    def _():
        pltpu.sync_copy(zeros_hbm.at[pl.ds(r0, RCHUNK)],
                        acc_sh.at[pl.ds(r0, RCHUNK)])

        @pl.when(sid == NS - 1)
        def _():
            pltpu.sync_copy(zeros_hbm.at[pl.ds(rtail0, ntail)],
                            acc_sh.at[pl.ds(rtail0, ntail)])

    plsc.subcore_barrier()

    def body(j, carry):
        base = wid * ET + j * K
        pltpu.sync_copy(src_hbm.at[pl.ds(base, K)], src_v)
        pltpu.sync_copy(dst_hbm.at[pl.ds(base, K)], dst_v)
        pltpu.async_copy(y_hbm.at[src_v], rows_v, sem).wait()
        pltpu.sync_copy(rows_v, acc_sh.at[dst_v], add=True)
        return carry

    lax.fori_loop(0, NB, body, 0)
    plsc.subcore_barrier()
    pltpu.sync_copy(acc_sh.at[pl.ds(r0, RCHUNK)],
                    out_hbm.at[cid].at[pl.ds(r0, RCHUNK)])

    @pl.when(sid == NS - 1)
    def _():
        pltpu.sync_copy(acc_sh.at[pl.ds(rtail0, ntail)],
                        out_hbm.at[cid].at[pl.ds(rtail0, ntail)])


_RB = 2000
_NRB = N // _RB


def _tc_prep_body(hist_ref, x_ref, w_ref, y_ref, dinv_ref):
    h = hist_ref[...]
    deg = h[:, 0:1] + h[:, 1:2] + 1.0
    dinv = lax.rsqrt(deg)
    dinv_ref[...] = dinv
    y_ref[...] = dinv * jnp.dot(x_ref[...], w_ref[...],
                                preferred_element_type=jnp.float32)


def _tc_prep(hist2, x, w1):
    return pl.pallas_call(
        _tc_prep_body,
        grid=(_NRB,),
        in_specs=[
            pl.BlockSpec((_RB, NC), lambda j: (j, 0)),
            pl.BlockSpec((_RB, D), lambda j: (j, 0)),
            pl.BlockSpec((D, D), lambda j: (0, 0)),
        ],
        out_specs=[
            pl.BlockSpec((_RB, D), lambda j: (j, 0)),
            pl.BlockSpec((_RB, 1), lambda j: (j, 0)),
        ],
        out_shape=[
            jax.ShapeDtypeStruct((N, D), jnp.float32),
            jax.ShapeDtypeStruct((N, 1), jnp.float32),
        ],
    )(hist2, x, w1)


def _tc_mid_body(acc_ref, dinv_ref, b_ref, w_ref, y_ref):
    dinv = dinv_ref[...]
    h = dinv * (acc_ref[0] + acc_ref[1]) + b_ref[...]
    h = jnp.maximum(h, 0.0)
    y_ref[...] = dinv * jnp.dot(h, w_ref[...],
                                preferred_element_type=jnp.float32)


def _tc_mid(acc2, dinv, b, w_next):
    return pl.pallas_call(
        _tc_mid_body,
        grid=(_NRB,),
        in_specs=[
            pl.BlockSpec((NC, _RB, D), lambda j: (0, j, 0)),
            pl.BlockSpec((_RB, 1), lambda j: (j, 0)),
            pl.BlockSpec((1, D), lambda j: (0, 0)),
            pl.BlockSpec((D, D), lambda j: (0, 0)),
        ],
        out_specs=pl.BlockSpec((_RB, D), lambda j: (j, 0)),
        out_shape=jax.ShapeDtypeStruct((N, D), jnp.float32),
    )(acc2, dinv, b.reshape(1, D), w_next)


def _tc_final_body(acc_ref, dinv_ref, b_ref, batch_ref, out_ref, sums, cnt):
    j = pl.program_id(0)
    h3 = dinv_ref[...] * (acc_ref[0] + acc_ref[1]) + b_ref[...]
    batch = batch_ref[0, 0, :].reshape(1, _RB)
    gid = lax.broadcasted_iota(jnp.int32, (G, _RB), 0)
    p = (gid == batch).astype(jnp.float32)

    @pl.when(j == 0)
    def _():
        sums[...] = jnp.zeros_like(sums)
        cnt[...] = jnp.zeros_like(cnt)

    sums[...] += jnp.dot(p, h3, preferred_element_type=jnp.float32)
    cnt[...] += jnp.sum(p, axis=1, keepdims=True)

    @pl.when(j == _NRB - 1)
    def _():
        out_ref[...] = sums[...] / jnp.maximum(cnt[...], 1.0)


def _tc_final(acc2, dinv, b3, batch):
    return pl.pallas_call(
        _tc_final_body,
        grid=(_NRB,),
        in_specs=[
            pl.BlockSpec((NC, _RB, D), lambda j: (0, j, 0)),
            pl.BlockSpec((_RB, 1), lambda j: (j, 0)),
            pl.BlockSpec((1, D), lambda j: (0, 0)),
            pl.BlockSpec((1, 1, _RB), lambda j: (j, 0, 0)),
        ],
        out_specs=pl.BlockSpec((G, D), lambda j: (0, 0)),
        out_shape=jax.ShapeDtypeStruct((G, D), jnp.float32),
        scratch_shapes=[
            pltpu.VMEM((G, D), jnp.float32),
            pltpu.VMEM((G, 1), jnp.float32),
        ],
    )(acc2, dinv, b3.reshape(1, D), batch.reshape(_NRB, 1, _RB))


def kernel(x, edge_index, batch, W1, b1, W2, b2, W3, b3):
    src = edge_index[0]
    dst = edge_index[1]
    zeros_nd = jnp.zeros((N, D), jnp.float32)
    zeros_np = jnp.zeros((NP,), jnp.float32)
    ones_k = jnp.ones((K,), jnp.float32)

    hist2 = _sc_degree(dst, zeros_np, ones_k)
    y1, dinv = _tc_prep(hist2[:, :N].T, x, W1)
    acc1 = _sc_scatter(y1, zeros_nd, src, dst)
    y2 = _tc_mid(acc1, dinv, b1, W2)
    acc2 = _sc_scatter(y2, zeros_nd, src, dst)
    y3 = _tc_mid(acc2, dinv, b2, W3)
    acc3 = _sc_scatter(y3, zeros_nd, src, dst)
    return _tc_final(acc3, dinv, b3, batch)


# speedup vs baseline: 20.0585x; 1.7487x over previous
import functools

import jax
import jax.numpy as jnp
from jax import lax
from jax.experimental import pallas as pl
from jax.experimental.pallas import tpu as pltpu
from jax.experimental.pallas import tpu_sc as plsc

N = 10000
E = 320000
D = 128
G = 64

NC = 2
NS = 16
NW = NC * NS
ET = E // NW
K = 80
NB = ET // K
RCHUNK = 624

_mesh = plsc.VectorSubcoreMesh(core_axis_name="c", subcore_axis_name="s")


NP = 10240


@functools.partial(
    pl.kernel,
    mesh=_mesh,
    out_type=jax.ShapeDtypeStruct((NC, NP), jnp.float32),
    scratch_types=[
        pltpu.VMEM_SHARED((NP,), jnp.float32),
        pltpu.VMEM((K,), jnp.int32),
        pltpu.VMEM((K,), jnp.float32),
    ],
)
def _sc_degree(dst_hbm, zeros_hbm, ones_hbm, out_hbm, hist_sh, dst_v, ones_v):
    cid = lax.axis_index("c")
    sid = lax.axis_index("s")
    wid = cid * NS + sid
    r0 = sid * 640
    pltpu.sync_copy(zeros_hbm.at[pl.ds(r0, 640)],
                    hist_sh.at[pl.ds(r0, 640)])
    pltpu.sync_copy(ones_hbm, ones_v)
    plsc.subcore_barrier()

    def body(j, carry):
        base = wid * ET + j * K
        pltpu.sync_copy(dst_hbm.at[pl.ds(base, K)], dst_v)
        pltpu.sync_copy(ones_v, hist_sh.at[dst_v], add=True)
        return carry

    lax.fori_loop(0, NB, body, 0)
    plsc.subcore_barrier()
    pltpu.sync_copy(hist_sh.at[pl.ds(r0, 640)],
                    out_hbm.at[cid].at[pl.ds(r0, 640)])


NSLOT = 2


@functools.partial(
    pl.kernel,
    mesh=_mesh,
    out_type=jax.ShapeDtypeStruct((NC, N, D), jnp.float32),
    scratch_types=[
        pltpu.VMEM_SHARED((N, D), jnp.float32),
        pltpu.VMEM((ET,), jnp.int32),
        pltpu.VMEM((NB, K), jnp.int32),
        pltpu.VMEM((NSLOT, K, D), jnp.float32),
    ] + [pltpu.SemaphoreType.DMA] * (2 * NSLOT),
)
def _sc_scatter(y_hbm, zeros_hbm, src_hbm, dst3_hbm, out_hbm,
                acc_sh, srcv, dstv, rows,
                g0, g1, s0, s1):
    gs = [g0, g1]
    ss = [s0, s1]
    cid = lax.axis_index("c")
    sid = lax.axis_index("s")
    wid = cid * NS + sid
    r0 = sid * RCHUNK
    rtail0 = NS * RCHUNK
    ntail = N - rtail0

    pltpu.sync_copy(src_hbm.at[pl.ds(wid * ET, ET)], srcv)
    pltpu.sync_copy(dst3_hbm.at[wid], dstv)

    @pl.when(cid == 0)
    def _():
        pltpu.sync_copy(y_hbm.at[pl.ds(r0, RCHUNK)],
                        acc_sh.at[pl.ds(r0, RCHUNK)])

        @pl.when(sid == NS - 1)
        def _():
            pltpu.sync_copy(y_hbm.at[pl.ds(rtail0, ntail)],
                            acc_sh.at[pl.ds(rtail0, ntail)])

    @pl.when(cid != 0)
    def _():
        pltpu.sync_copy(zeros_hbm.at[pl.ds(r0, RCHUNK)],
                        acc_sh.at[pl.ds(r0, RCHUNK)])

        @pl.when(sid == NS - 1)
        def _():
            pltpu.sync_copy(zeros_hbm.at[pl.ds(rtail0, ntail)],
                            acc_sh.at[pl.ds(rtail0, ntail)])

    def g_start(j, b):
        pltpu.async_copy(y_hbm.at[srcv.at[pl.ds(j * K, K)]], rows.at[b], gs[b])

    def g_wait(j, b):
        pltpu.make_async_copy(y_hbm.at[srcv.at[pl.ds(j * K, K)]],
                              rows.at[b], gs[b]).wait()

    def s_start(j, b):
        pltpu.async_copy(rows.at[b], acc_sh.at[dstv.at[j]], ss[b], add=True)

    def s_wait(j, b):
        pltpu.make_async_copy(rows.at[b], acc_sh.at[dstv.at[j]], ss[b]).wait()

    for b in range(NSLOT - 1):
        g_start(b, b)

    plsc.subcore_barrier()

    def body(jo, carry):
        for b in range(NSLOT):
            j = jo * NSLOT + b

            @pl.when(j < NB)
            def _():
                g_wait(j, b)
                s_start(j, b)

            @pl.when(jnp.logical_and(j >= 1, j - 1 < NB))
            def _():
                s_wait(j - 1, (b + NSLOT - 1) % NSLOT)

            @pl.when(j + NSLOT - 1 < NB)
            def _():
                g_start(j + NSLOT - 1, (b + NSLOT - 1) % NSLOT)

        return carry

    nouter = (NB + 2 * NSLOT - 1) // NSLOT
    lax.fori_loop(0, nouter, body, 0)
    plsc.subcore_barrier()
    pltpu.sync_copy(acc_sh.at[pl.ds(r0, RCHUNK)],
                    out_hbm.at[cid].at[pl.ds(r0, RCHUNK)])

    @pl.when(sid == NS - 1)
    def _():
        pltpu.sync_copy(acc_sh.at[pl.ds(rtail0, ntail)],
                        out_hbm.at[cid].at[pl.ds(rtail0, ntail)])


_RB = 2000
_NRB = N // _RB


def _tc_prep_body(hist_ref, x_ref, w_ref, y_ref, dinv_ref):
    h = hist_ref[...]
    deg = h[:, 0:1] + h[:, 1:2] + 1.0
    dinv = lax.rsqrt(deg)
    dinv_ref[...] = dinv
    y_ref[...] = dinv * jnp.dot(x_ref[...], w_ref[...],
                                preferred_element_type=jnp.float32)


def _tc_prep(hist2, x, w1):
    return pl.pallas_call(
        _tc_prep_body,
        grid=(_NRB,),
        in_specs=[
            pl.BlockSpec((_RB, NC), lambda j: (j, 0)),
            pl.BlockSpec((_RB, D), lambda j: (j, 0)),
            pl.BlockSpec((D, D), lambda j: (0, 0)),
        ],
        out_specs=[
            pl.BlockSpec((_RB, D), lambda j: (j, 0)),
            pl.BlockSpec((_RB, 1), lambda j: (j, 0)),
        ],
        out_shape=[
            jax.ShapeDtypeStruct((N, D), jnp.float32),
            jax.ShapeDtypeStruct((N, 1), jnp.float32),
        ],
    )(hist2, x, w1)


def _tc_mid_body(acc_ref, dinv_ref, b_ref, w_ref, y_ref):
    dinv = dinv_ref[...]
    h = dinv * (acc_ref[0] + acc_ref[1]) + b_ref[...]
    h = jnp.maximum(h, 0.0)
    y_ref[...] = dinv * jnp.dot(h, w_ref[...],
                                preferred_element_type=jnp.float32)


def _tc_mid(acc2, dinv, b, w_next):
    return pl.pallas_call(
        _tc_mid_body,
        grid=(_NRB,),
        in_specs=[
            pl.BlockSpec((NC, _RB, D), lambda j: (0, j, 0)),
            pl.BlockSpec((_RB, 1), lambda j: (j, 0)),
            pl.BlockSpec((1, D), lambda j: (0, 0)),
            pl.BlockSpec((D, D), lambda j: (0, 0)),
        ],
        out_specs=pl.BlockSpec((_RB, D), lambda j: (j, 0)),
        out_shape=jax.ShapeDtypeStruct((N, D), jnp.float32),
    )(acc2, dinv, b.reshape(1, D), w_next)


def _tc_final_body(acc_ref, dinv_ref, b_ref, batch_ref, out_ref, sums, cnt):
    j = pl.program_id(0)
    h3 = dinv_ref[...] * (acc_ref[0] + acc_ref[1]) + b_ref[...]
    batch = batch_ref[0, 0, :].reshape(1, _RB)
    gid = lax.broadcasted_iota(jnp.int32, (G, _RB), 0)
    p = (gid == batch).astype(jnp.float32)

    @pl.when(j == 0)
    def _():
        sums[...] = jnp.zeros_like(sums)
        cnt[...] = jnp.zeros_like(cnt)

    sums[...] += jnp.dot(p, h3, preferred_element_type=jnp.float32)
    cnt[...] += jnp.sum(p, axis=1, keepdims=True)

    @pl.when(j == _NRB - 1)
    def _():
        out_ref[...] = sums[...] / jnp.maximum(cnt[...], 1.0)


def _tc_final(acc2, dinv, b3, batch):
    return pl.pallas_call(
        _tc_final_body,
        grid=(_NRB,),
        in_specs=[
            pl.BlockSpec((NC, _RB, D), lambda j: (0, j, 0)),
            pl.BlockSpec((_RB, 1), lambda j: (j, 0)),
            pl.BlockSpec((1, D), lambda j: (0, 0)),
            pl.BlockSpec((1, 1, _RB), lambda j: (j, 0, 0)),
        ],
        out_specs=pl.BlockSpec((G, D), lambda j: (0, 0)),
        out_shape=jax.ShapeDtypeStruct((G, D), jnp.float32),
        scratch_shapes=[
            pltpu.VMEM((G, D), jnp.float32),
            pltpu.VMEM((G, 1), jnp.float32),
        ],
    )(acc2, dinv, b3.reshape(1, D), batch.reshape(_NRB, 1, _RB))


def kernel(x, edge_index, batch, W1, b1, W2, b2, W3, b3):
    src = edge_index[0]
    dst = edge_index[1]
    zeros_nd = jnp.zeros((N, D), jnp.float32)
    zeros_np = jnp.zeros((NP,), jnp.float32)
    ones_k = jnp.ones((K,), jnp.float32)

    dst3 = dst.reshape(NW, NB, K)

    hist2 = _sc_degree(dst, zeros_np, ones_k)
    y1, dinv = _tc_prep(hist2[:, :N].T, x, W1)
    acc1 = _sc_scatter(y1, zeros_nd, src, dst3)
    y2 = _tc_mid(acc1, dinv, b1, W2)
    acc2 = _sc_scatter(y2, zeros_nd, src, dst3)
    y3 = _tc_mid(acc2, dinv, b2, W3)
    acc3 = _sc_scatter(y3, zeros_nd, src, dst3)
    return _tc_final(acc3, dinv, b3, batch)


# speedup vs baseline: 32.0458x; 1.5976x over previous
import functools

import jax
import jax.numpy as jnp
from jax import lax
from jax.experimental import pallas as pl
from jax.experimental.pallas import tpu as pltpu
from jax.experimental.pallas import tpu_sc as plsc

N = 10000
E = 320000
D = 128
G = 64

NC = 2
NS = 16
NW = NC * NS
ET = E // NW
K = 80
NB = ET // K
RCHUNK = 624

_mesh = plsc.VectorSubcoreMesh(core_axis_name="c", subcore_axis_name="s")


NP = 10240


@functools.partial(
    pl.kernel,
    mesh=_mesh,
    out_type=jax.ShapeDtypeStruct((NC, NP), jnp.float32),
    scratch_types=[
        pltpu.VMEM_SHARED((NP,), jnp.float32),
        pltpu.VMEM((NB, K), jnp.int32),
        pltpu.VMEM((K,), jnp.float32),
        pltpu.SemaphoreType.DMA,
    ],
)
def _sc_degree(dst3_hbm, zeros_hbm, ones_hbm, out_hbm, hist_sh, dstv, ones_v,
               sem):
    cid = lax.axis_index("c")
    sid = lax.axis_index("s")
    wid = cid * NS + sid
    r0 = sid * 640
    pltpu.sync_copy(zeros_hbm.at[pl.ds(r0, 640)],
                    hist_sh.at[pl.ds(r0, 640)])
    pltpu.sync_copy(dst3_hbm.at[wid], dstv)
    pltpu.sync_copy(ones_hbm, ones_v)
    plsc.subcore_barrier()

    def fire(j, carry):
        pltpu.async_copy(ones_v, hist_sh.at[dstv.at[j]], sem, add=True)
        return carry

    lax.fori_loop(0, NB, fire, 0)

    def drain(j, carry):
        pltpu.make_async_copy(ones_v, hist_sh.at[dstv.at[j]], sem).wait()
        return carry

    lax.fori_loop(0, NB, drain, 0)
    plsc.subcore_barrier()
    pltpu.sync_copy(hist_sh.at[pl.ds(r0, 640)],
                    out_hbm.at[cid].at[pl.ds(r0, 640)])


NSLOT = 3


@functools.partial(
    pl.kernel,
    mesh=_mesh,
    out_type=jax.ShapeDtypeStruct((NC, N, D), jnp.float32),
    scratch_types=[
        pltpu.VMEM_SHARED((N, D), jnp.float32),
        pltpu.VMEM((ET,), jnp.int32),
        pltpu.VMEM((ET,), jnp.int32),
        pltpu.VMEM((NSLOT, K, D), jnp.float32),
    ] + [pltpu.SemaphoreType.DMA] * (2 * NSLOT),
)
def _sc_scatter(y_hbm, zeros_hbm, src_hbm, dst_hbm, out_hbm,
                acc_sh, srcv, dstv, rows,
                g0, g1, g2, s0, s1, s2):
    gs = [g0, g1, g2]
    ss = [s0, s1, s2]
    cid = lax.axis_index("c")
    sid = lax.axis_index("s")
    wid = cid * NS + sid
    r0 = sid * RCHUNK
    rtail0 = NS * RCHUNK
    ntail = N - rtail0

    pltpu.sync_copy(src_hbm.at[pl.ds(wid * ET, ET)], srcv)
    pltpu.sync_copy(dst_hbm.at[pl.ds(wid * ET, ET)], dstv)

    @pl.when(cid == 0)
    def _():
        pltpu.sync_copy(y_hbm.at[pl.ds(r0, RCHUNK)],
                        acc_sh.at[pl.ds(r0, RCHUNK)])

        @pl.when(sid == NS - 1)
        def _():
            pltpu.sync_copy(y_hbm.at[pl.ds(rtail0, ntail)],
                            acc_sh.at[pl.ds(rtail0, ntail)])

    @pl.when(cid != 0)
    def _():
        pltpu.sync_copy(zeros_hbm.at[pl.ds(r0, RCHUNK)],
                        acc_sh.at[pl.ds(r0, RCHUNK)])

        @pl.when(sid == NS - 1)
        def _():
            pltpu.sync_copy(zeros_hbm.at[pl.ds(rtail0, ntail)],
                            acc_sh.at[pl.ds(rtail0, ntail)])

    def g_start(j, b):
        pltpu.async_copy(y_hbm.at[srcv.at[pl.ds(j * K, K)]], rows.at[b], gs[b])

    def g_wait(j, b):
        pltpu.make_async_copy(y_hbm.at[srcv.at[pl.ds(j * K, K)]],
                              rows.at[b], gs[b]).wait()

    def s_start(j, b):
        pltpu.async_copy(rows.at[b], acc_sh.at[dstv.at[pl.ds(j * K, K)]],
                         ss[b], add=True)

    def s_wait(j, b):
        pltpu.make_async_copy(rows.at[b], acc_sh.at[dstv.at[pl.ds(j * K, K)]],
                              ss[b]).wait()

    for b in range(NSLOT - 1):
        g_start(b, b)

    plsc.subcore_barrier()

    def body(jo, carry):
        for b in range(NSLOT):
            j = jo * NSLOT + b

            @pl.when(j < NB)
            def _():
                g_wait(j, b)
                s_start(j, b)

            @pl.when(jnp.logical_and(j >= 1, j - 1 < NB))
            def _():
                s_wait(j - 1, (b + NSLOT - 1) % NSLOT)

            @pl.when(j + NSLOT - 1 < NB)
            def _():
                g_start(j + NSLOT - 1, (b + NSLOT - 1) % NSLOT)

        return carry

    nouter = (NB + 2 * NSLOT - 1) // NSLOT
    lax.fori_loop(0, nouter, body, 0)
    plsc.subcore_barrier()
    pltpu.sync_copy(acc_sh.at[pl.ds(r0, RCHUNK)],
                    out_hbm.at[cid].at[pl.ds(r0, RCHUNK)])

    @pl.when(sid == NS - 1)
    def _():
        pltpu.sync_copy(acc_sh.at[pl.ds(rtail0, ntail)],
                        out_hbm.at[cid].at[pl.ds(rtail0, ntail)])


_RB = 2000
_NRB = N // _RB


def _tc_prep_body(hist_ref, x_ref, w_ref, y_ref, dinv_ref):
    h = hist_ref[...]
    deg = h[:, 0:1] + h[:, 1:2] + 1.0
    dinv = lax.rsqrt(deg)
    dinv_ref[...] = dinv
    y_ref[...] = dinv * jnp.dot(x_ref[...], w_ref[...],
                                preferred_element_type=jnp.float32)


def _tc_prep(hist2, x, w1):
    return pl.pallas_call(
        _tc_prep_body,
        grid=(_NRB,),
        in_specs=[
            pl.BlockSpec((_RB, NC), lambda j: (j, 0)),
            pl.BlockSpec((_RB, D), lambda j: (j, 0)),
            pl.BlockSpec((D, D), lambda j: (0, 0)),
        ],
        out_specs=[
            pl.BlockSpec((_RB, D), lambda j: (j, 0)),
            pl.BlockSpec((_RB, 1), lambda j: (j, 0)),
        ],
        out_shape=[
            jax.ShapeDtypeStruct((N, D), jnp.float32),
            jax.ShapeDtypeStruct((N, 1), jnp.float32),
        ],
    )(hist2, x, w1)


def _tc_mid_body(acc_ref, dinv_ref, b_ref, w_ref, y_ref):
    dinv = dinv_ref[...]
    h = dinv * (acc_ref[0] + acc_ref[1]) + b_ref[...]
    h = jnp.maximum(h, 0.0)
    y_ref[...] = dinv * jnp.dot(h, w_ref[...],
                                preferred_element_type=jnp.float32)


def _tc_mid(acc2, dinv, b, w_next):
    return pl.pallas_call(
        _tc_mid_body,
        grid=(_NRB,),
        in_specs=[
            pl.BlockSpec((NC, _RB, D), lambda j: (0, j, 0)),
            pl.BlockSpec((_RB, 1), lambda j: (j, 0)),
            pl.BlockSpec((1, D), lambda j: (0, 0)),
            pl.BlockSpec((D, D), lambda j: (0, 0)),
        ],
        out_specs=pl.BlockSpec((_RB, D), lambda j: (j, 0)),
        out_shape=jax.ShapeDtypeStruct((N, D), jnp.float32),
    )(acc2, dinv, b.reshape(1, D), w_next)


def _tc_final_body(acc_ref, dinv_ref, b_ref, batch_ref, out_ref, sums, cnt):
    j = pl.program_id(0)
    h3 = dinv_ref[...] * (acc_ref[0] + acc_ref[1]) + b_ref[...]
    batch = batch_ref[0, 0, :].reshape(1, _RB)
    gid = lax.broadcasted_iota(jnp.int32, (G, _RB), 0)
    p = (gid == batch).astype(jnp.float32)

    @pl.when(j == 0)
    def _():
        sums[...] = jnp.zeros_like(sums)
        cnt[...] = jnp.zeros_like(cnt)

    sums[...] += jnp.dot(p, h3, preferred_element_type=jnp.float32)
    cnt[...] += jnp.sum(p, axis=1, keepdims=True)

    @pl.when(j == _NRB - 1)
    def _():
        out_ref[...] = sums[...] / jnp.maximum(cnt[...], 1.0)


def _tc_final(acc2, dinv, b3, batch):
    return pl.pallas_call(
        _tc_final_body,
        grid=(_NRB,),
        in_specs=[
            pl.BlockSpec((NC, _RB, D), lambda j: (0, j, 0)),
            pl.BlockSpec((_RB, 1), lambda j: (j, 0)),
            pl.BlockSpec((1, D), lambda j: (0, 0)),
            pl.BlockSpec((1, 1, _RB), lambda j: (j, 0, 0)),
        ],
        out_specs=pl.BlockSpec((G, D), lambda j: (0, 0)),
        out_shape=jax.ShapeDtypeStruct((G, D), jnp.float32),
        scratch_shapes=[
            pltpu.VMEM((G, D), jnp.float32),
            pltpu.VMEM((G, 1), jnp.float32),
        ],
    )(acc2, dinv, b3.reshape(1, D), batch.reshape(_NRB, 1, _RB))


def kernel(x, edge_index, batch, W1, b1, W2, b2, W3, b3):
    src = edge_index[0]
    dst = edge_index[1]
    zeros_nd = jnp.zeros((N, D), jnp.float32)
    zeros_np = jnp.zeros((NP,), jnp.float32)
    ones_k = jnp.ones((K,), jnp.float32)

    dst3 = dst.reshape(NW, NB, K)

    hist2 = _sc_degree(dst3, zeros_np, ones_k)
    y1, dinv = _tc_prep(hist2[:, :N].T, x, W1)
    acc1 = _sc_scatter(y1, zeros_nd, src, dst)
    y2 = _tc_mid(acc1, dinv, b1, W2)
    acc2 = _sc_scatter(y2, zeros_nd, src, dst)
    y3 = _tc_mid(acc2, dinv, b2, W3)
    acc3 = _sc_scatter(y3, zeros_nd, src, dst)
    return _tc_final(acc3, dinv, b3, batch)


# speedup vs baseline: 33.2272x; 1.0369x over previous
import functools

import jax
import jax.numpy as jnp
from jax import lax
from jax.experimental import pallas as pl
from jax.experimental.pallas import tpu as pltpu
from jax.experimental.pallas import tpu_sc as plsc

N = 10000
E = 320000
D = 128
G = 64

NC = 2
NS = 16
NW = NC * NS
ET = E // NW
K = 40
NB = ET // K
RCHUNK = 624

_mesh = plsc.VectorSubcoreMesh(core_axis_name="c", subcore_axis_name="s")


NP = 10240


@functools.partial(
    pl.kernel,
    mesh=_mesh,
    out_type=jax.ShapeDtypeStruct((NC, NP), jnp.float32),
    scratch_types=[
        pltpu.VMEM_SHARED((NP,), jnp.float32),
        pltpu.VMEM((NB, K), jnp.int32),
        pltpu.VMEM((K,), jnp.float32),
        pltpu.SemaphoreType.DMA,
    ],
)
def _sc_degree(dst3_hbm, zeros_hbm, ones_hbm, out_hbm, hist_sh, dstv, ones_v,
               sem):
    cid = lax.axis_index("c")
    sid = lax.axis_index("s")
    wid = cid * NS + sid
    r0 = sid * 640
    pltpu.sync_copy(zeros_hbm.at[pl.ds(r0, 640)],
                    hist_sh.at[pl.ds(r0, 640)])
    pltpu.sync_copy(dst3_hbm.at[wid], dstv)
    pltpu.sync_copy(ones_hbm, ones_v)
    plsc.subcore_barrier()

    def fire(j, carry):
        pltpu.async_copy(ones_v, hist_sh.at[dstv.at[j]], sem, add=True)
        return carry

    lax.fori_loop(0, NB, fire, 0)

    def drain(j, carry):
        pltpu.make_async_copy(ones_v, hist_sh.at[dstv.at[j]], sem).wait()
        return carry

    lax.fori_loop(0, NB, drain, 0)
    plsc.subcore_barrier()
    pltpu.sync_copy(hist_sh.at[pl.ds(r0, 640)],
                    out_hbm.at[cid].at[pl.ds(r0, 640)])


NSLOT = 6


@functools.partial(
    pl.kernel,
    mesh=_mesh,
    out_type=jax.ShapeDtypeStruct((NC, N, D), jnp.float32),
    scratch_types=[
        pltpu.VMEM_SHARED((N, D), jnp.float32),
        pltpu.VMEM((ET,), jnp.int32),
        pltpu.VMEM((ET,), jnp.int32),
        pltpu.VMEM((NSLOT, K, D), jnp.float32),
    ] + [pltpu.SemaphoreType.DMA] * (2 * NSLOT),
)
def _sc_scatter(y_hbm, zeros_hbm, src_hbm, dst_hbm, out_hbm,
                acc_sh, srcv, dstv, rows, *sems):
    gs = list(sems[:NSLOT])
    ss = list(sems[NSLOT:])
    cid = lax.axis_index("c")
    sid = lax.axis_index("s")
    wid = cid * NS + sid
    r0 = sid * RCHUNK
    rtail0 = NS * RCHUNK
    ntail = N - rtail0

    pltpu.sync_copy(src_hbm.at[pl.ds(wid * ET, ET)], srcv)
    pltpu.sync_copy(dst_hbm.at[pl.ds(wid * ET, ET)], dstv)

    @pl.when(cid == 0)
    def _():
        pltpu.sync_copy(y_hbm.at[pl.ds(r0, RCHUNK)],
                        acc_sh.at[pl.ds(r0, RCHUNK)])

        @pl.when(sid == NS - 1)
        def _():
            pltpu.sync_copy(y_hbm.at[pl.ds(rtail0, ntail)],
                            acc_sh.at[pl.ds(rtail0, ntail)])

    @pl.when(cid != 0)
    def _():
        pltpu.sync_copy(zeros_hbm.at[pl.ds(r0, RCHUNK)],
                        acc_sh.at[pl.ds(r0, RCHUNK)])

        @pl.when(sid == NS - 1)
        def _():
            pltpu.sync_copy(zeros_hbm.at[pl.ds(rtail0, ntail)],
                            acc_sh.at[pl.ds(rtail0, ntail)])

    def g_start(j, b):
        pltpu.async_copy(y_hbm.at[srcv.at[pl.ds(j * K, K)]], rows.at[b], gs[b])

    def g_wait(j, b):
        pltpu.make_async_copy(y_hbm.at[srcv.at[pl.ds(j * K, K)]],
                              rows.at[b], gs[b]).wait()

    def s_start(j, b):
        pltpu.async_copy(rows.at[b], acc_sh.at[dstv.at[pl.ds(j * K, K)]],
                         ss[b], add=True)

    def s_wait(j, b):
        pltpu.make_async_copy(rows.at[b], acc_sh.at[dstv.at[pl.ds(j * K, K)]],
                              ss[b]).wait()

    for b in range(NSLOT - 1):
        g_start(b, b)

    plsc.subcore_barrier()

    def body(jo, carry):
        for b in range(NSLOT):
            j = jo * NSLOT + b

            @pl.when(j < NB)
            def _():
                g_wait(j, b)
                s_start(j, b)

            @pl.when(jnp.logical_and(j >= 1, j - 1 < NB))
            def _():
                s_wait(j - 1, (b + NSLOT - 1) % NSLOT)

            @pl.when(j + NSLOT - 1 < NB)
            def _():
                g_start(j + NSLOT - 1, (b + NSLOT - 1) % NSLOT)

        return carry

    nouter = (NB + 2 * NSLOT - 1) // NSLOT
    lax.fori_loop(0, nouter, body, 0)
    plsc.subcore_barrier()
    pltpu.sync_copy(acc_sh.at[pl.ds(r0, RCHUNK)],
                    out_hbm.at[cid].at[pl.ds(r0, RCHUNK)])

    @pl.when(sid == NS - 1)
    def _():
        pltpu.sync_copy(acc_sh.at[pl.ds(rtail0, ntail)],
                        out_hbm.at[cid].at[pl.ds(rtail0, ntail)])


_RB = 2000
_NRB = N // _RB


def _tc_prep_body(hist_ref, x_ref, w_ref, y_ref, dinv_ref):
    h = hist_ref[...]
    deg = h[:, 0:1] + h[:, 1:2] + 1.0
    dinv = lax.rsqrt(deg)
    dinv_ref[...] = dinv
    y_ref[...] = dinv * jnp.dot(x_ref[...], w_ref[...],
                                preferred_element_type=jnp.float32)


def _tc_prep(hist2, x, w1):
    return pl.pallas_call(
        _tc_prep_body,
        grid=(_NRB,),
        in_specs=[
            pl.BlockSpec((_RB, NC), lambda j: (j, 0)),
            pl.BlockSpec((_RB, D), lambda j: (j, 0)),
            pl.BlockSpec((D, D), lambda j: (0, 0)),
        ],
        out_specs=[
            pl.BlockSpec((_RB, D), lambda j: (j, 0)),
            pl.BlockSpec((_RB, 1), lambda j: (j, 0)),
        ],
        out_shape=[
            jax.ShapeDtypeStruct((N, D), jnp.float32),
            jax.ShapeDtypeStruct((N, 1), jnp.float32),
        ],
    )(hist2, x, w1)


def _tc_mid_body(acc_ref, dinv_ref, b_ref, w_ref, y_ref):
    dinv = dinv_ref[...]
    h = dinv * (acc_ref[0] + acc_ref[1]) + b_ref[...]
    h = jnp.maximum(h, 0.0)
    y_ref[...] = dinv * jnp.dot(h, w_ref[...],
                                preferred_element_type=jnp.float32)


def _tc_mid(acc2, dinv, b, w_next):
    return pl.pallas_call(
        _tc_mid_body,
        grid=(_NRB,),
        in_specs=[
            pl.BlockSpec((NC, _RB, D), lambda j: (0, j, 0)),
            pl.BlockSpec((_RB, 1), lambda j: (j, 0)),
            pl.BlockSpec((1, D), lambda j: (0, 0)),
            pl.BlockSpec((D, D), lambda j: (0, 0)),
        ],
        out_specs=pl.BlockSpec((_RB, D), lambda j: (j, 0)),
        out_shape=jax.ShapeDtypeStruct((N, D), jnp.float32),
    )(acc2, dinv, b.reshape(1, D), w_next)


def _tc_final_body(acc_ref, dinv_ref, b_ref, batch_ref, out_ref, sums, cnt):
    j = pl.program_id(0)
    h3 = dinv_ref[...] * (acc_ref[0] + acc_ref[1]) + b_ref[...]
    batch = batch_ref[0, 0, :].reshape(1, _RB)
    gid = lax.broadcasted_iota(jnp.int32, (G, _RB), 0)
    p = (gid == batch).astype(jnp.float32)

    @pl.when(j == 0)
    def _():
        sums[...] = jnp.zeros_like(sums)
        cnt[...] = jnp.zeros_like(cnt)

    sums[...] += jnp.dot(p, h3, preferred_element_type=jnp.float32)
    cnt[...] += jnp.sum(p, axis=1, keepdims=True)

    @pl.when(j == _NRB - 1)
    def _():
        out_ref[...] = sums[...] / jnp.maximum(cnt[...], 1.0)


def _tc_final(acc2, dinv, b3, batch):
    return pl.pallas_call(
        _tc_final_body,
        grid=(_NRB,),
        in_specs=[
            pl.BlockSpec((NC, _RB, D), lambda j: (0, j, 0)),
            pl.BlockSpec((_RB, 1), lambda j: (j, 0)),
            pl.BlockSpec((1, D), lambda j: (0, 0)),
            pl.BlockSpec((1, 1, _RB), lambda j: (j, 0, 0)),
        ],
        out_specs=pl.BlockSpec((G, D), lambda j: (0, 0)),
        out_shape=jax.ShapeDtypeStruct((G, D), jnp.float32),
        scratch_shapes=[
            pltpu.VMEM((G, D), jnp.float32),
            pltpu.VMEM((G, 1), jnp.float32),
        ],
    )(acc2, dinv, b3.reshape(1, D), batch.reshape(_NRB, 1, _RB))


def kernel(x, edge_index, batch, W1, b1, W2, b2, W3, b3):
    src = edge_index[0]
    dst = edge_index[1]
    zeros_nd = jnp.zeros((N, D), jnp.float32)
    zeros_np = jnp.zeros((NP,), jnp.float32)
    ones_k = jnp.ones((K,), jnp.float32)

    dst3 = dst.reshape(NW, NB, K)

    hist2 = _sc_degree(dst3, zeros_np, ones_k)
    y1, dinv = _tc_prep(hist2[:, :N].T, x, W1)
    acc1 = _sc_scatter(y1, zeros_nd, src, dst)
    y2 = _tc_mid(acc1, dinv, b1, W2)
    acc2 = _sc_scatter(y2, zeros_nd, src, dst)
    y3 = _tc_mid(acc2, dinv, b2, W3)
    acc3 = _sc_scatter(y3, zeros_nd, src, dst)
    return _tc_final(acc3, dinv, b3, batch)


# speedup vs baseline: 33.5147x; 1.0087x over previous
import functools

import jax
import jax.numpy as jnp
from jax import lax
from jax.experimental import pallas as pl
from jax.experimental.pallas import tpu as pltpu
from jax.experimental.pallas import tpu_sc as plsc

N = 10000
E = 320000
D = 128
G = 64

NC = 2
NS = 16
NW = NC * NS
ET = E // NW
K = 40
NB = ET // K
RCHUNK = 624

_mesh = plsc.VectorSubcoreMesh(core_axis_name="c", subcore_axis_name="s")


NP = 10240


@functools.partial(
    pl.kernel,
    mesh=_mesh,
    out_type=jax.ShapeDtypeStruct((NC, NP), jnp.float32),
    scratch_types=[
        pltpu.VMEM_SHARED((NP,), jnp.float32),
        pltpu.VMEM((NB, K), jnp.int32),
        pltpu.VMEM((K,), jnp.float32),
        pltpu.SemaphoreType.DMA,
    ],
)
def _sc_degree(dst3_hbm, zeros_hbm, ones_hbm, out_hbm, hist_sh, dstv, ones_v,
               sem):
    cid = lax.axis_index("c")
    sid = lax.axis_index("s")
    wid = cid * NS + sid
    r0 = sid * 640
    pltpu.sync_copy(zeros_hbm.at[pl.ds(r0, 640)],
                    hist_sh.at[pl.ds(r0, 640)])
    pltpu.sync_copy(dst3_hbm.at[wid], dstv)
    pltpu.sync_copy(ones_hbm, ones_v)
    plsc.subcore_barrier()

    def fire(j, carry):
        pltpu.async_copy(ones_v, hist_sh.at[dstv.at[j]], sem, add=True)
        return carry

    lax.fori_loop(0, NB, fire, 0)

    def drain(j, carry):
        pltpu.make_async_copy(ones_v, hist_sh.at[dstv.at[j]], sem).wait()
        return carry

    lax.fori_loop(0, NB, drain, 0)
    plsc.subcore_barrier()
    pltpu.sync_copy(hist_sh.at[pl.ds(r0, 640)],
                    out_hbm.at[cid].at[pl.ds(r0, 640)])


NSLOT = 6


@functools.partial(
    pl.kernel,
    mesh=_mesh,
    out_type=jax.ShapeDtypeStruct((NC, N, D), jnp.float32),
    scratch_types=[
        pltpu.VMEM_SHARED((N, D), jnp.float32),
        pltpu.VMEM((ET,), jnp.int32),
        pltpu.VMEM((ET,), jnp.int32),
        pltpu.VMEM((NSLOT, K, D), jnp.float32),
    ] + [pltpu.SemaphoreType.DMA] * (2 * NSLOT),
)
def _sc_scatter(y_hbm, zeros_hbm, src_hbm, dst_hbm, out_hbm,
                acc_sh, srcv, dstv, rows, *sems):
    gs = list(sems[:NSLOT])
    ss = list(sems[NSLOT:])
    cid = lax.axis_index("c")
    sid = lax.axis_index("s")
    wid = cid * NS + sid
    r0 = sid * RCHUNK
    rtail0 = NS * RCHUNK
    ntail = N - rtail0

    pltpu.sync_copy(src_hbm.at[pl.ds(wid * ET, ET)], srcv)
    pltpu.sync_copy(dst_hbm.at[pl.ds(wid * ET, ET)], dstv)

    @pl.when(cid == 0)
    def _():
        pltpu.sync_copy(y_hbm.at[pl.ds(r0, RCHUNK)],
                        acc_sh.at[pl.ds(r0, RCHUNK)])

        @pl.when(sid == NS - 1)
        def _():
            pltpu.sync_copy(y_hbm.at[pl.ds(rtail0, ntail)],
                            acc_sh.at[pl.ds(rtail0, ntail)])

    @pl.when(cid != 0)
    def _():
        pltpu.sync_copy(zeros_hbm.at[pl.ds(r0, RCHUNK)],
                        acc_sh.at[pl.ds(r0, RCHUNK)])

        @pl.when(sid == NS - 1)
        def _():
            pltpu.sync_copy(zeros_hbm.at[pl.ds(rtail0, ntail)],
                            acc_sh.at[pl.ds(rtail0, ntail)])

    def g_start(j, b):
        pltpu.async_copy(y_hbm.at[srcv.at[pl.ds(j * K, K)]], rows.at[b], gs[b])

    def g_wait(j, b):
        pltpu.make_async_copy(y_hbm.at[srcv.at[pl.ds(j * K, K)]],
                              rows.at[b], gs[b]).wait()

    def s_start(j, b):
        pltpu.async_copy(rows.at[b], acc_sh.at[dstv.at[pl.ds(j * K, K)]],
                         ss[b], add=True)

    def s_wait(j, b):
        pltpu.make_async_copy(rows.at[b], acc_sh.at[dstv.at[pl.ds(j * K, K)]],
                              ss[b]).wait()

    for b in range(NSLOT - 1):
        g_start(b, b)

    plsc.subcore_barrier()

    def body(jo, carry):
        for b in range(NSLOT):
            j = jo * NSLOT + b

            @pl.when(jnp.logical_and(j >= 1, j - 1 < NB))
            def _():
                s_wait(j - 1, (b + NSLOT - 1) % NSLOT)

            @pl.when(j + NSLOT - 1 < NB)
            def _():
                g_start(j + NSLOT - 1, (b + NSLOT - 1) % NSLOT)

            @pl.when(j < NB)
            def _():
                g_wait(j, b)
                s_start(j, b)

        return carry

    nouter = (NB + 2 * NSLOT - 1) // NSLOT
    lax.fori_loop(0, nouter, body, 0)
    plsc.subcore_barrier()
    pltpu.sync_copy(acc_sh.at[pl.ds(r0, RCHUNK)],
                    out_hbm.at[cid].at[pl.ds(r0, RCHUNK)])

    @pl.when(sid == NS - 1)
    def _():
        pltpu.sync_copy(acc_sh.at[pl.ds(rtail0, ntail)],
                        out_hbm.at[cid].at[pl.ds(rtail0, ntail)])


_RB = 2000
_NRB = N // _RB


def _tc_prep_body(hist_ref, x_ref, w_ref, y_ref, dinv_ref):
    h = hist_ref[...]
    deg = h[:, 0:1] + h[:, 1:2] + 1.0
    dinv = lax.rsqrt(deg)
    dinv_ref[...] = dinv
    y_ref[...] = dinv * jnp.dot(x_ref[...], w_ref[...],
                                preferred_element_type=jnp.float32)


def _tc_prep(hist2, x, w1):
    return pl.pallas_call(
        _tc_prep_body,
        grid=(_NRB,),
        in_specs=[
            pl.BlockSpec((_RB, NC), lambda j: (j, 0)),
            pl.BlockSpec((_RB, D), lambda j: (j, 0)),
            pl.BlockSpec((D, D), lambda j: (0, 0)),
        ],
        out_specs=[
            pl.BlockSpec((_RB, D), lambda j: (j, 0)),
            pl.BlockSpec((_RB, 1), lambda j: (j, 0)),
        ],
        out_shape=[
            jax.ShapeDtypeStruct((N, D), jnp.float32),
            jax.ShapeDtypeStruct((N, 1), jnp.float32),
        ],
    )(hist2, x, w1)


def _tc_mid_body(acc_ref, dinv_ref, b_ref, w_ref, y_ref):
    dinv = dinv_ref[...]
    h = dinv * (acc_ref[0] + acc_ref[1]) + b_ref[...]
    h = jnp.maximum(h, 0.0)
    y_ref[...] = dinv * jnp.dot(h, w_ref[...],
                                preferred_element_type=jnp.float32)


def _tc_mid(acc2, dinv, b, w_next):
    return pl.pallas_call(
        _tc_mid_body,
        grid=(_NRB,),
        in_specs=[
            pl.BlockSpec((NC, _RB, D), lambda j: (0, j, 0)),
            pl.BlockSpec((_RB, 1), lambda j: (j, 0)),
            pl.BlockSpec((1, D), lambda j: (0, 0)),
            pl.BlockSpec((D, D), lambda j: (0, 0)),
        ],
        out_specs=pl.BlockSpec((_RB, D), lambda j: (j, 0)),
        out_shape=jax.ShapeDtypeStruct((N, D), jnp.float32),
    )(acc2, dinv, b.reshape(1, D), w_next)


def _tc_final_body(acc_ref, dinv_ref, b_ref, batch_ref, out_ref, sums, cnt):
    j = pl.program_id(0)
    h3 = dinv_ref[...] * (acc_ref[0] + acc_ref[1]) + b_ref[...]
    batch = batch_ref[0, 0, :].reshape(1, _RB)
    gid = lax.broadcasted_iota(jnp.int32, (G, _RB), 0)
    p = (gid == batch).astype(jnp.float32)

    @pl.when(j == 0)
    def _():
        sums[...] = jnp.zeros_like(sums)
        cnt[...] = jnp.zeros_like(cnt)

    sums[...] += jnp.dot(p, h3, preferred_element_type=jnp.float32)
    cnt[...] += jnp.sum(p, axis=1, keepdims=True)

    @pl.when(j == _NRB - 1)
    def _():
        out_ref[...] = sums[...] / jnp.maximum(cnt[...], 1.0)


def _tc_final(acc2, dinv, b3, batch):
    return pl.pallas_call(
        _tc_final_body,
        grid=(_NRB,),
        in_specs=[
            pl.BlockSpec((NC, _RB, D), lambda j: (0, j, 0)),
            pl.BlockSpec((_RB, 1), lambda j: (j, 0)),
            pl.BlockSpec((1, D), lambda j: (0, 0)),
            pl.BlockSpec((1, 1, _RB), lambda j: (j, 0, 0)),
        ],
        out_specs=pl.BlockSpec((G, D), lambda j: (0, 0)),
        out_shape=jax.ShapeDtypeStruct((G, D), jnp.float32),
        scratch_shapes=[
            pltpu.VMEM((G, D), jnp.float32),
            pltpu.VMEM((G, 1), jnp.float32),
        ],
    )(acc2, dinv, b3.reshape(1, D), batch.reshape(_NRB, 1, _RB))


def kernel(x, edge_index, batch, W1, b1, W2, b2, W3, b3):
    src = edge_index[0]
    dst = edge_index[1]
    zeros_nd = jnp.zeros((N, D), jnp.float32)
    zeros_np = jnp.zeros((NP,), jnp.float32)
    ones_k = jnp.ones((K,), jnp.float32)

    dst3 = dst.reshape(NW, NB, K)

    hist2 = _sc_degree(dst3, zeros_np, ones_k)
    y1, dinv = _tc_prep(hist2[:, :N].T, x, W1)
    acc1 = _sc_scatter(y1, zeros_nd, src, dst)
    y2 = _tc_mid(acc1, dinv, b1, W2)
    acc2 = _sc_scatter(y2, zeros_nd, src, dst)
    y3 = _tc_mid(acc2, dinv, b2, W3)
    acc3 = _sc_scatter(y3, zeros_nd, src, dst)
    return _tc_final(acc3, dinv, b3, batch)
